# baseline (speedup 1.0000x reference)
import functools

import jax
import jax.numpy as jnp
from jax import lax
from jax.experimental import pallas as pl
from jax.experimental.pallas import tpu as pltpu
from jax.experimental.pallas import tpu_sc as plsc

N_ROWS = 1024
N_FEAT = 128
N_PIDS = 100000
N_CQ = 5000
SCALE = 30.0
IGNORE = 5554
LOG2E = 1.4426950408889634
LN2 = 0.6931471805599453

TILE_LUT = 10000


def _lane_partial_sums(e, acc, ncols):
    nfull = ncols // N_FEAT
    part = e[:, 0:N_FEAT]
    for k in range(1, nfull):
        part = part + e[:, k * N_FEAT:(k + 1) * N_FEAT]
    acc = acc + part
    rem = ncols - nfull * N_FEAT
    if rem:
        tail = acc[:, 0:rem] + e[:, nfull * N_FEAT:ncols]
        acc = jnp.concatenate([tail, acc[:, rem:N_FEAT]], axis=1)
    return acc


def _sumexp_body(xs_ref, w_ref, s_ref):
    i = pl.program_id(0)
    t = lax.dot_general(
        xs_ref[...], w_ref[...].astype(jnp.bfloat16),
        (((1,), (1,)), ((), ())),
        preferred_element_type=jnp.float32,
    )
    base = jnp.where(i == 0, jnp.zeros_like(s_ref), s_ref[...])
    s_ref[...] = _lane_partial_sums(jnp.exp2(t), base, TILE_LUT)


def _sumexp_lut(xs, lut):
    return pl.pallas_call(
        _sumexp_body,
        grid=(N_PIDS // TILE_LUT,),
        in_specs=[
            pl.BlockSpec((N_ROWS, N_FEAT), lambda i: (0, 0)),
            pl.BlockSpec((TILE_LUT, N_FEAT), lambda i: (i, 0)),
        ],
        out_specs=pl.BlockSpec((N_ROWS, N_FEAT), lambda i: (0, 0)),
        out_shape=jax.ShapeDtypeStruct((N_ROWS, N_FEAT), jnp.float32),
    )(xs, lut)


_NC = 1
_NW = 16 * _NC
_BPW = N_ROWS // _NW
_L = 16


@functools.lru_cache(maxsize=1)
def _make_sc_gather():
    mesh = plsc.VectorSubcoreMesh(
        core_axis_name="c", subcore_axis_name="s", num_cores=_NC)

    @functools.partial(
        pl.kernel,
        mesh=mesh,
        out_type=[jax.ShapeDtypeStruct((N_ROWS, N_FEAT), jnp.float32),
                  jax.ShapeDtypeStruct((N_ROWS,), jnp.float32)],
        scratch_types=[
            pltpu.VMEM((_BPW,), jnp.int32),
            pltpu.VMEM((_BPW, N_FEAT), jnp.float32),
            pltpu.VMEM((_BPW,), jnp.float32),
            pltpu.SemaphoreType.DMA,
        ],
    )
    def gather_k(table_hbm, tgt_hbm, g_hbm, vm_hbm,
                 idx_v, rows_v, vm_v, sem):
        wid = lax.axis_index("s") * _NC + lax.axis_index("c")
        base = wid * _BPW
        pltpu.sync_copy(
            tgt_hbm.at[wid // 2, pl.ds((wid % 2) * _BPW, _BPW)], idx_v)
        for g in range(_BPW // _L):
            lbl = idx_v[pl.ds(g * _L, _L)] - 1
            idx_v[pl.ds(g * _L, _L)] = jnp.maximum(lbl, 0)
            ok = jnp.logical_and(lbl >= 0, lbl != IGNORE)
            vm_v[pl.ds(g * _L, _L)] = jnp.where(ok, 1.0, 0.0)
        pltpu.async_copy(table_hbm.at[idx_v], rows_v, sem).wait()
        pltpu.sync_copy(rows_v, g_hbm.at[pl.ds(base, _BPW)])
        pltpu.sync_copy(vm_v, vm_hbm.at[pl.ds(base, _BPW)])

    return gather_k


def _combine_body(xs_ref, cq_ref, g_ref, vm_ref, sa_ref, out_ref):
    t = lax.dot_general(
        xs_ref[...], cq_ref[...].astype(jnp.bfloat16),
        (((1,), (1,)), ((), ())),
        preferred_element_type=jnp.float32,
    )
    s128 = _lane_partial_sums(jnp.exp2(t), sa_ref[...], N_CQ)
    s = jnp.sum(s128, axis=1, keepdims=True)
    xsf = xs_ref[...].astype(jnp.float32)
    picked = LN2 * jnp.sum(xsf * g_ref[...], axis=1, keepdims=True)
    nll = jnp.log(s) - picked
    vm1 = vm_ref[...].reshape(1, N_ROWS)
    num = lax.dot_general(vm1, nll, (((1,), (0,)), ((), ())),
                          preferred_element_type=jnp.float32)
    den = jnp.maximum(jnp.sum(vm1, axis=1, keepdims=True), 1.0)
    out_ref[...] = num / den


def _combine(xs, cq, g, vm, sa):
    return pl.pallas_call(
        _combine_body,
        out_shape=jax.ShapeDtypeStruct((1, 1), jnp.float32),
    )(xs, cq, g, vm, sa)


def kernel(inputs, roi_label, ious, lut, cq):
    xs = (inputs * (SCALE * LOG2E)).astype(jnp.bfloat16)

    g, vm = _make_sc_gather()(lut, roi_label)
    s_lut = _sumexp_lut(xs, lut)
    loss = _combine(xs, cq, g, vm, s_lut)
    return jnp.nan_to_num(loss.reshape(()))


# speedup vs baseline: 1.0233x; 1.0233x over previous
import functools

import jax
import jax.numpy as jnp
from jax import lax
from jax.experimental import pallas as pl
from jax.experimental.pallas import tpu as pltpu
from jax.experimental.pallas import tpu_sc as plsc

N_ROWS = 1024
N_FEAT = 128
N_PIDS = 100000
N_CQ = 5000
SCALE = 30.0
IGNORE = 5554
LOG2E = 1.4426950408889634
LN2 = 0.6931471805599453

TILE_LUT = 10000


def _lane_partial_sums(e, acc, ncols):
    nfull = ncols // N_FEAT
    part = e[:, 0:N_FEAT]
    for k in range(1, nfull):
        part = part + e[:, k * N_FEAT:(k + 1) * N_FEAT]
    acc = acc + part
    rem = ncols - nfull * N_FEAT
    if rem:
        tail = acc[:, 0:rem] + e[:, nfull * N_FEAT:ncols]
        acc = jnp.concatenate([tail, acc[:, rem:N_FEAT]], axis=1)
    return acc


def _sumexp_body(x_ref, w_ref, s_ref):
    i = pl.program_id(0)
    xb = (x_ref[...] * (SCALE * LOG2E)).astype(jnp.bfloat16)
    t = lax.dot_general(
        xb, w_ref[...].astype(jnp.bfloat16),
        (((1,), (1,)), ((), ())),
        preferred_element_type=jnp.float32,
    )
    base = jnp.where(i == 0, jnp.zeros_like(s_ref), s_ref[...])
    s_ref[...] = _lane_partial_sums(jnp.exp2(t), base, TILE_LUT)


def _sumexp_lut(x, lut):
    return pl.pallas_call(
        _sumexp_body,
        grid=(N_PIDS // TILE_LUT,),
        in_specs=[
            pl.BlockSpec((N_ROWS, N_FEAT), lambda i: (0, 0)),
            pl.BlockSpec((TILE_LUT, N_FEAT), lambda i: (i, 0)),
        ],
        out_specs=pl.BlockSpec((N_ROWS, N_FEAT), lambda i: (0, 0)),
        out_shape=jax.ShapeDtypeStruct((N_ROWS, N_FEAT), jnp.float32),
    )(x, lut)


_NC = 1
_NW = 16 * _NC
_BPW = N_ROWS // _NW
_L = 16


@functools.lru_cache(maxsize=1)
def _make_sc_gather():
    mesh = plsc.VectorSubcoreMesh(
        core_axis_name="c", subcore_axis_name="s", num_cores=_NC)

    @functools.partial(
        pl.kernel,
        mesh=mesh,
        out_type=[jax.ShapeDtypeStruct((N_ROWS, N_FEAT), jnp.float32),
                  jax.ShapeDtypeStruct((N_ROWS,), jnp.float32)],
        scratch_types=[
            pltpu.VMEM((_BPW,), jnp.int32),
            pltpu.VMEM((_BPW, N_FEAT), jnp.float32),
            pltpu.VMEM((_BPW,), jnp.float32),
            pltpu.SemaphoreType.DMA,
        ],
    )
    def gather_k(table_hbm, tgt_hbm, g_hbm, vm_hbm,
                 idx_v, rows_v, vm_v, sem):
        wid = lax.axis_index("s") * _NC + lax.axis_index("c")
        base = wid * _BPW
        pltpu.sync_copy(
            tgt_hbm.at[wid // 2, pl.ds((wid % 2) * _BPW, _BPW)], idx_v)
        for g in range(_BPW // _L):
            lbl = idx_v[pl.ds(g * _L, _L)] - 1
            idx_v[pl.ds(g * _L, _L)] = jnp.maximum(lbl, 0)
            ok = jnp.logical_and(lbl >= 0, lbl != IGNORE)
            vm_v[pl.ds(g * _L, _L)] = jnp.where(ok, 1.0, 0.0)
        pltpu.async_copy(table_hbm.at[idx_v], rows_v, sem).wait()
        pltpu.sync_copy(rows_v, g_hbm.at[pl.ds(base, _BPW)])
        pltpu.sync_copy(vm_v, vm_hbm.at[pl.ds(base, _BPW)])

    return gather_k


def _combine_body(x_ref, cq_ref, g_ref, vm_ref, sa_ref, out_ref):
    x = x_ref[...]
    xb = (x * (SCALE * LOG2E)).astype(jnp.bfloat16)
    t = lax.dot_general(
        xb, cq_ref[...].astype(jnp.bfloat16),
        (((1,), (1,)), ((), ())),
        preferred_element_type=jnp.float32,
    )
    s128 = _lane_partial_sums(jnp.exp2(t), sa_ref[...], N_CQ)
    s = jnp.sum(s128, axis=1, keepdims=True)
    picked = SCALE * jnp.sum(x * g_ref[...], axis=1, keepdims=True)
    nll = jnp.log(s) - picked
    vm1 = vm_ref[...].reshape(1, N_ROWS)
    num = lax.dot_general(vm1, nll, (((1,), (0,)), ((), ())),
                          preferred_element_type=jnp.float32)
    den = jnp.maximum(jnp.sum(vm1, axis=1, keepdims=True), 1.0)
    out_ref[...] = num / den


def _combine(x, cq, g, vm, sa):
    return pl.pallas_call(
        _combine_body,
        out_shape=jax.ShapeDtypeStruct((1, 1), jnp.float32),
    )(x, cq, g, vm, sa)


def kernel(inputs, roi_label, ious, lut, cq):
    g, vm = _make_sc_gather()(lut, roi_label)
    s_lut = _sumexp_lut(inputs, lut)
    loss = _combine(inputs, cq, g, vm, s_lut)
    return jnp.nan_to_num(loss.reshape(()))
